# baseline (speedup 1.0000x reference)
import jax
import jax.numpy as jnp
from jax import lax
from jax.experimental import pallas as pl
from jax.experimental.pallas import tpu as pltpu

_K = 16
_N = 16384
_C = 20
_B = 8
_ROWS = 1024

_NN = (((1,), (0,)), ((), ()))


def _dot(a, b):
    return lax.dot_general(a, b, _NN, preferred_element_type=jnp.float32)


def _loss_body(pred_r, pred_b, cr, cTb, sr, sTb, out_sum, out_cnt):
    b = pl.program_id(0)
    r = pl.program_id(1)
    R = _ROWS
    n = _N // _B

    sq_r = jnp.sum(cr[...] * cr[...], axis=1, keepdims=True)
    sq_b = jnp.sum(cTb[...] * cTb[...], axis=0, keepdims=True)
    cross = _dot(cr[...], cTb[...])
    d2 = sq_r + sq_b - 2.0 * cross

    thr = jnp.full((R, 1), -1e30, jnp.float32)
    for _ in range(_K):
        thr = jnp.min(jnp.where(d2 > thr, d2, 1e30), axis=1, keepdims=True)
    lt_f = jnp.where(d2 < thr, 1.0, 0.0)
    eq_f = jnp.where(d2 == thr, 1.0, 0.0)
    ones_n = jnp.ones((n, 1), jnp.float32)
    strict = _dot(lt_f, ones_n)
    mult = _dot(eq_f, ones_n)
    frac = jnp.clip((float(_K) - strict) / jnp.maximum(mult, 1.0), 0.0, 1.0)
    w = lt_f + eq_f * frac

    iota_r = lax.broadcasted_iota(jnp.int32, (1, _C), 1).astype(jnp.float32)
    iota_c = lax.broadcasted_iota(jnp.int32, (_C, 1), 0).astype(jnp.float32)
    oh_r = jnp.where(sr[...] == iota_r, 1.0, 0.0)
    oh_b = jnp.where(iota_c == sTb[...], 1.0, 0.0)
    eq = _dot(oh_r, oh_b)
    m = w * eq

    pr = pred_r[...]
    er = jnp.exp(pr - jnp.max(pr, axis=1, keepdims=True))
    probs_r = er / jnp.sum(er, axis=1, keepdims=True)
    pb = pred_b[...]
    eb = jnp.exp(pb - jnp.max(pb, axis=1, keepdims=True))
    probs_b = eb / jnp.sum(eb, axis=1, keepdims=True)

    sqp_r = jnp.sum(probs_r * probs_r, axis=1, keepdims=True)
    sqp_b = jnp.sum(probs_b * probs_b, axis=1, keepdims=True)
    z = jnp.concatenate([probs_b, sqp_b, ones_n], axis=1)
    o = _dot(m, z)
    rows = (o[:, _C:_C + 1] + sqp_r * o[:, _C + 1:_C + 2]
            - 2.0 * jnp.sum(probs_r * o[:, :_C], axis=1, keepdims=True))
    local_sum = jnp.sum(rows)
    local_cnt = jnp.sum(o[:, _C + 1:_C + 2])

    @pl.when((b == 0) & (r == 0))
    def _():
        out_sum[0, 0] = 0.0
        out_cnt[0, 0] = 0.0

    out_sum[0, 0] += local_sum
    out_cnt[0, 0] += local_cnt


def kernel(pred, coord, offset, segment):
    n = _N // _B
    r_tiles = n // _ROWS

    coord_p = jnp.concatenate(
        [coord, jnp.zeros((_N, 1), jnp.float32)], axis=1)
    coord_t = coord_p.T
    segf = segment.astype(jnp.float32)
    seg_r = segf.reshape(_N, 1)
    seg_t = segf.reshape(1, _N)

    grid = (_B, r_tiles)
    out_sum, out_cnt = pl.pallas_call(
        _loss_body,
        grid=grid,
        in_specs=[
            pl.BlockSpec((_ROWS, _C), lambda b, r: (b * r_tiles + r, 0)),
            pl.BlockSpec((n, _C), lambda b, r: (b, 0)),
            pl.BlockSpec((_ROWS, 4), lambda b, r: (b * r_tiles + r, 0)),
            pl.BlockSpec((4, n), lambda b, r: (0, b)),
            pl.BlockSpec((_ROWS, 1), lambda b, r: (b * r_tiles + r, 0)),
            pl.BlockSpec((1, n), lambda b, r: (0, b)),
        ],
        out_specs=[
            pl.BlockSpec(memory_space=pltpu.SMEM),
            pl.BlockSpec(memory_space=pltpu.SMEM),
        ],
        out_shape=[
            jax.ShapeDtypeStruct((1, 1), jnp.float32),
            jax.ShapeDtypeStruct((1, 1), jnp.float32),
        ],
    )(pred, pred, coord_p, coord_t, seg_r, seg_t)

    total = out_sum[0, 0]
    count = jnp.maximum(out_cnt[0, 0], 1.0)
    return total / count


# speedup vs baseline: 1.1109x; 1.1109x over previous
import jax
import jax.numpy as jnp
from jax import lax
from jax.experimental import pallas as pl
from jax.experimental.pallas import tpu as pltpu

_K = 16
_N = 16384
_C = 20
_B = 8
_ROWS = 1024

_NN = (((1,), (0,)), ((), ()))


def _dot(a, b):
    return lax.dot_general(a, b, _NN, preferred_element_type=jnp.float32)


def _loss_body(pred_r, predT_b, cr, cTb, sr, sTb, out_sum, out_cnt):
    b = pl.program_id(0)
    r = pl.program_id(1)

    sq_r = jnp.sum(cr[...] * cr[...], axis=1, keepdims=True)
    sq_b = jnp.sum(cTb[...] * cTb[...], axis=0, keepdims=True)
    cross = _dot(cr[...], cTb[...])
    d2 = sq_r + sq_b - 2.0 * cross

    R = d2.shape[0]
    thr = jnp.full((R, 1), -1e30, jnp.float32)
    for _ in range(_K):
        thr = jnp.min(jnp.where(d2 > thr, d2, 1e30), axis=1, keepdims=True)
    strict = jnp.sum((d2 < thr).astype(jnp.float32), axis=1, keepdims=True)
    mult = jnp.sum((d2 == thr).astype(jnp.float32), axis=1, keepdims=True)
    frac = jnp.clip((float(_K) - strict) / jnp.maximum(mult, 1.0), 0.0, 1.0)
    w = jnp.where(d2 < thr, 1.0, jnp.where(d2 == thr, frac, 0.0))

    pr = pred_r[...]
    er = jnp.exp(pr - jnp.max(pr, axis=1, keepdims=True))
    probs_r = er / jnp.sum(er, axis=1, keepdims=True)
    pb = predT_b[...]
    eb = jnp.exp(pb - jnp.max(pb, axis=0, keepdims=True))
    probs_b = eb / jnp.sum(eb, axis=0, keepdims=True)

    sqp_r = jnp.sum(probs_r * probs_r, axis=1, keepdims=True)
    sqp_b = jnp.sum(probs_b * probs_b, axis=0, keepdims=True)
    g = _dot(probs_r, probs_b)
    pd = sqp_r + sqp_b - 2.0 * g

    eq = (sr[...] == sTb[...]).astype(jnp.float32)
    m = w * eq
    local_sum = jnp.sum(m * pd)
    local_cnt = jnp.sum(m)

    @pl.when((b == 0) & (r == 0))
    def _():
        out_sum[0, 0] = 0.0
        out_cnt[0, 0] = 0.0

    out_sum[0, 0] += local_sum
    out_cnt[0, 0] += local_cnt


def kernel(pred, coord, offset, segment):
    n = _N // _B
    r_tiles = n // _ROWS

    coord_p = jnp.concatenate(
        [coord, jnp.zeros((_N, 1), jnp.float32)], axis=1)
    coord_t = coord_p.T
    segf = segment.astype(jnp.float32)
    seg_r = segf.reshape(_N, 1)
    seg_t = segf.reshape(1, _N)
    pred_t = pred.T

    grid = (_B, r_tiles)
    out_sum, out_cnt = pl.pallas_call(
        _loss_body,
        grid=grid,
        in_specs=[
            pl.BlockSpec((_ROWS, _C), lambda b, r: (b * r_tiles + r, 0)),
            pl.BlockSpec((_C, n), lambda b, r: (0, b)),
            pl.BlockSpec((_ROWS, 4), lambda b, r: (b * r_tiles + r, 0)),
            pl.BlockSpec((4, n), lambda b, r: (0, b)),
            pl.BlockSpec((_ROWS, 1), lambda b, r: (b * r_tiles + r, 0)),
            pl.BlockSpec((1, n), lambda b, r: (0, b)),
        ],
        out_specs=[
            pl.BlockSpec(memory_space=pltpu.SMEM),
            pl.BlockSpec(memory_space=pltpu.SMEM),
        ],
        out_shape=[
            jax.ShapeDtypeStruct((1, 1), jnp.float32),
            jax.ShapeDtypeStruct((1, 1), jnp.float32),
        ],
    )(pred, pred_t, coord_p, coord_t, seg_r, seg_t)

    total = out_sum[0, 0]
    count = jnp.maximum(out_cnt[0, 0], 1.0)
    return total / count


# speedup vs baseline: 1.1441x; 1.0299x over previous
import jax
import jax.numpy as jnp
from jax import lax
from jax.experimental import pallas as pl
from jax.experimental.pallas import tpu as pltpu

_K = 16
_N = 16384
_C = 20
_B = 8
_ROWS = 1024

_NN = (((1,), (0,)), ((), ()))


def _dot(a, b):
    return lax.dot_general(a, b, _NN, preferred_element_type=jnp.float32)


def _loss_body(pred_r, predT_b, cr, cTb, sr, sTb, out_sum, out_cnt):
    b = pl.program_id(0)
    r = pl.program_id(1)

    sq_r = jnp.sum(cr[...] * cr[...], axis=1, keepdims=True)
    sq_b = jnp.sum(cTb[...] * cTb[...], axis=0, keepdims=True)
    cross = _dot(cr[...], cTb[...])
    d2 = sq_r + sq_b - 2.0 * cross

    thr = jnp.min(d2, axis=1, keepdims=True)
    for _ in range(_K - 1):
        thr = jnp.min(jnp.where(d2 > thr, d2, 1e30), axis=1, keepdims=True)
    strict = jnp.sum((d2 < thr).astype(jnp.float32), axis=1, keepdims=True)
    mult = jnp.sum((d2 == thr).astype(jnp.float32), axis=1, keepdims=True)
    frac = jnp.clip((float(_K) - strict) / jnp.maximum(mult, 1.0), 0.0, 1.0)
    w = jnp.where(d2 < thr, 1.0, jnp.where(d2 == thr, frac, 0.0))

    pr = pred_r[...]
    er = jnp.exp(pr - jnp.max(pr, axis=1, keepdims=True))
    probs_r = er / jnp.sum(er, axis=1, keepdims=True)
    pb = predT_b[...]
    eb = jnp.exp(pb - jnp.max(pb, axis=0, keepdims=True))
    probs_b = eb / jnp.sum(eb, axis=0, keepdims=True)

    sqp_r = jnp.sum(probs_r * probs_r, axis=1, keepdims=True)
    sqp_b = jnp.sum(probs_b * probs_b, axis=0, keepdims=True)
    g = _dot(probs_r, probs_b)
    pd = sqp_r + sqp_b - 2.0 * g

    eq = (sr[...] == sTb[...]).astype(jnp.float32)
    m = w * eq
    local_sum = jnp.sum(m * pd)
    local_cnt = jnp.sum(m)

    @pl.when((b == 0) & (r == 0))
    def _():
        out_sum[0, 0] = 0.0
        out_cnt[0, 0] = 0.0

    out_sum[0, 0] += local_sum
    out_cnt[0, 0] += local_cnt


def kernel(pred, coord, offset, segment):
    n = _N // _B
    r_tiles = n // _ROWS

    coord_p = jnp.concatenate(
        [coord, jnp.zeros((_N, 1), jnp.float32)], axis=1)
    coord_t = coord_p.T
    segf = segment.astype(jnp.float32)
    seg_r = segf.reshape(_N, 1)
    seg_t = segf.reshape(1, _N)
    pred_t = pred.T

    grid = (_B, r_tiles)
    out_sum, out_cnt = pl.pallas_call(
        _loss_body,
        grid=grid,
        in_specs=[
            pl.BlockSpec((_ROWS, _C), lambda b, r: (b * r_tiles + r, 0)),
            pl.BlockSpec((_C, n), lambda b, r: (0, b)),
            pl.BlockSpec((_ROWS, 4), lambda b, r: (b * r_tiles + r, 0)),
            pl.BlockSpec((4, n), lambda b, r: (0, b)),
            pl.BlockSpec((_ROWS, 1), lambda b, r: (b * r_tiles + r, 0)),
            pl.BlockSpec((1, n), lambda b, r: (0, b)),
        ],
        out_specs=[
            pl.BlockSpec(memory_space=pltpu.SMEM),
            pl.BlockSpec(memory_space=pltpu.SMEM),
        ],
        out_shape=[
            jax.ShapeDtypeStruct((1, 1), jnp.float32),
            jax.ShapeDtypeStruct((1, 1), jnp.float32),
        ],
    )(pred, pred_t, coord_p, coord_t, seg_r, seg_t)

    total = out_sum[0, 0]
    count = jnp.maximum(out_cnt[0, 0], 1.0)
    return total / count


# speedup vs baseline: 1.3131x; 1.1477x over previous
import jax
import jax.numpy as jnp
from jax import lax
from jax.experimental import pallas as pl
from jax.experimental.pallas import tpu as pltpu

_K = 16
_N = 16384
_C = 20
_B = 8
_ROWS = 1024

_NN = (((1,), (0,)), ((), ()))


def _dot(a, b):
    return lax.dot_general(a, b, _NN, preferred_element_type=jnp.float32)


def _loss_body(pred_r, predT_b, cr, cTb, sr, sTb, out_sum, out_cnt):
    b = pl.program_id(0)
    r = pl.program_id(1)

    sq_r = jnp.sum(cr[...] * cr[...], axis=1, keepdims=True)
    sq_b = jnp.sum(cTb[...] * cTb[...], axis=0, keepdims=True)
    cross = _dot(cr[...], cTb[...])
    d2 = sq_r + sq_b - 2.0 * cross

    thr = jnp.min(d2, axis=1, keepdims=True)
    for _ in range(_K - 1):
        thr = jnp.min(jnp.where(d2 > thr, d2, 1e30), axis=1, keepdims=True)

    pr = pred_r[...]
    er = jnp.exp(pr - jnp.max(pr, axis=1, keepdims=True))
    probs_r = er / jnp.sum(er, axis=1, keepdims=True)
    pb = predT_b[...]
    eb = jnp.exp(pb - jnp.max(pb, axis=0, keepdims=True))
    probs_b = eb / jnp.sum(eb, axis=0, keepdims=True)

    sqp_r = jnp.sum(probs_r * probs_r, axis=1, keepdims=True)
    sqp_b = jnp.sum(probs_b * probs_b, axis=0, keepdims=True)
    g = _dot(probs_r, probs_b)
    pd = sqp_r + sqp_b - 2.0 * g

    m = jnp.where((d2 <= thr) & (sr[...] == sTb[...]), 1.0, 0.0)
    local_sum = jnp.sum(m * pd)
    local_cnt = jnp.sum(m)

    @pl.when((b == 0) & (r == 0))
    def _():
        out_sum[0, 0] = 0.0
        out_cnt[0, 0] = 0.0

    out_sum[0, 0] += local_sum
    out_cnt[0, 0] += local_cnt


def kernel(pred, coord, offset, segment):
    n = _N // _B
    r_tiles = n // _ROWS

    coord_p = jnp.concatenate(
        [coord, jnp.zeros((_N, 1), jnp.float32)], axis=1)
    coord_t = coord_p.T
    segf = segment.astype(jnp.float32)
    seg_r = segf.reshape(_N, 1)
    seg_t = segf.reshape(1, _N)
    pred_t = pred.T

    grid = (_B, r_tiles)
    out_sum, out_cnt = pl.pallas_call(
        _loss_body,
        grid=grid,
        in_specs=[
            pl.BlockSpec((_ROWS, _C), lambda b, r: (b * r_tiles + r, 0)),
            pl.BlockSpec((_C, n), lambda b, r: (0, b)),
            pl.BlockSpec((_ROWS, 4), lambda b, r: (b * r_tiles + r, 0)),
            pl.BlockSpec((4, n), lambda b, r: (0, b)),
            pl.BlockSpec((_ROWS, 1), lambda b, r: (b * r_tiles + r, 0)),
            pl.BlockSpec((1, n), lambda b, r: (0, b)),
        ],
        out_specs=[
            pl.BlockSpec(memory_space=pltpu.SMEM),
            pl.BlockSpec(memory_space=pltpu.SMEM),
        ],
        out_shape=[
            jax.ShapeDtypeStruct((1, 1), jnp.float32),
            jax.ShapeDtypeStruct((1, 1), jnp.float32),
        ],
    )(pred, pred_t, coord_p, coord_t, seg_r, seg_t)

    total = out_sum[0, 0]
    count = jnp.maximum(out_cnt[0, 0], 1.0)
    return total / count
